# baseline (speedup 1.0000x reference)
import jax
import jax.numpy as jnp
from jax.experimental import pallas as pl
from jax.experimental.pallas import tpu as pltpu

NUM_LAYERS = 2
NHEAD = 12
D_MODEL = 768
D_FFN = 1024
HEAD_DIM = D_MODEL // NHEAD
SEQ = 2048
TILE = 256
NTILES = SEQ // TILE
GW = 256
GRP = D_MODEL // GW


def _ln(x, g, b, eps=1e-5):
    mu = jnp.mean(x, axis=-1, keepdims=True)
    xc = x - mu
    var = jnp.mean(xc * xc, axis=-1, keepdims=True)
    return xc * jax.lax.rsqrt(var + eps) * g + b


def _bf(t):
    return t.astype(jnp.bfloat16)


def _dot(a, b):
    return jax.lax.dot_general(_bf(a), _bf(b), (((1,), (0,)), ((), ())),
                               preferred_element_type=jnp.float32)


def _dot_tn(a, b):
    return jax.lax.dot_general(_bf(a), _bf(b), (((0,), (0,)), ((), ())),
                               preferred_element_type=jnp.float32)


def _feat(t):
    return jnp.where(t > 0, t + 1.0, jnp.exp(t))


def _block_kernel(x_ref, Wq_ref, bq_ref, Wk_ref, bk_ref, Wv_ref, bv_ref,
                  Wo_ref, bo_ref, ln1g_ref, ln1b_ref, W1_ref, b1_ref,
                  W2_ref, b2_ref, ln2g_ref, ln2b_ref, lnfg_ref, lnfb_ref,
                  out_ref):
    i = pl.program_id(0)

    @pl.when(i == 0)
    def _():
        out_ref[...] = x_ref[...]

    def pass_a(t, carry):
        KV, Ksum = carry
        xt = _bf(out_ref[pl.ds(t * TILE, TILE), :])
        Kt = _feat(_dot(xt, Wk_ref[0]) + bk_ref[0])
        vt = _bf(_dot(xt, Wv_ref[0]) + bv_ref[0])
        Kt16 = _bf(Kt)
        KV = [KV[g] + _dot_tn(Kt16[:, g * GW:(g + 1) * GW],
                              vt[:, g * GW:(g + 1) * GW])
              for g in range(GRP)]
        return KV, Ksum + jnp.sum(Kt, axis=0, keepdims=True)

    KV0 = [jnp.zeros((GW, GW), jnp.float32) for _ in range(GRP)]
    Ks0 = jnp.zeros((1, D_MODEL), jnp.float32)
    carry = (KV0, Ks0)
    for t in range(NTILES):
        carry = pass_a(t, carry)
    KV, Ksum = carry

    r = jax.lax.broadcasted_iota(jnp.int32, (GW, GW), 0)
    c = jax.lax.broadcasted_iota(jnp.int32, (GW, GW), 1)
    gmask = r // HEAD_DIM == c // HEAD_DIM
    KVm = [_bf(jnp.where(gmask, KV[g], 0.0)) for g in range(GRP)]
    hd = jax.lax.broadcasted_iota(jnp.int32, (D_MODEL, NHEAD), 0)
    hh = jax.lax.broadcasted_iota(jnp.int32, (D_MODEL, NHEAD), 1)
    Bh = (hd // HEAD_DIM == hh).astype(jnp.bfloat16)

    def pass_b(t):
        xt = out_ref[pl.ds(t * TILE, TILE), :]
        Qt = _feat(_dot(xt, Wq_ref[0]) + bq_ref[0])
        Qt16 = _bf(Qt)
        num = jnp.concatenate(
            [_dot(Qt16[:, g * GW:(g + 1) * GW], KVm[g]) for g in range(GRP)],
            axis=1)
        den_h = _dot(Qt * Ksum, Bh)
        den = _dot(den_h, Bh.T)
        at = num / (den + 1e-6)
        at = _dot(at, Wo_ref[0]) + bo_ref[0]
        ht = _ln(xt + at, ln1g_ref[0], ln1b_ref[0])
        yt = jnp.maximum(_dot(ht, W1_ref[0]) + b1_ref[0], 0.0)
        yt = _dot(yt, W2_ref[0]) + b2_ref[0]
        x2t = _ln(ht + yt, ln2g_ref[0], ln2b_ref[0])

        @pl.when(i == NUM_LAYERS - 1)
        def _():
            out_ref[pl.ds(t * TILE, TILE), :] = _ln(x2t, lnfg_ref[0],
                                                    lnfb_ref[0])

        @pl.when(i != NUM_LAYERS - 1)
        def _():
            out_ref[pl.ds(t * TILE, TILE), :] = x2t

    for t in range(NTILES):
        pass_b(t)


@jax.jit
def kernel(x, Wq, bq, Wk, bk, Wv, bv, Wo, bo, ln1_g, ln1_b, W1, b1, W2, b2,
           ln2_g, ln2_b, lnf_g, lnf_b):
    N, L, D = x.shape
    x2 = x.reshape(N * L, D)
    r2 = lambda t: t.reshape(NUM_LAYERS, 1, t.shape[-1])
    bq, bk, bv, bo = r2(bq), r2(bk), r2(bv), r2(bo)
    ln1_g, ln1_b, ln2_g, ln2_b = r2(ln1_g), r2(ln1_b), r2(ln2_g), r2(ln2_b)
    b1, b2 = r2(b1), r2(b2)
    lnf_g2 = lnf_g.reshape(1, D)
    lnf_b2 = lnf_b.reshape(1, D)

    full2 = lambda t: pl.BlockSpec(t.shape, lambda i: (0, 0))
    layer3 = lambda t: pl.BlockSpec((1,) + t.shape[1:], lambda i: (i, 0, 0))

    out = pl.pallas_call(
        _block_kernel,
        grid=(NUM_LAYERS,),
        in_specs=[
            full2(x2),
            layer3(Wq), layer3(bq), layer3(Wk), layer3(bk),
            layer3(Wv), layer3(bv), layer3(Wo), layer3(bo),
            layer3(ln1_g), layer3(ln1_b),
            layer3(W1), layer3(b1), layer3(W2), layer3(b2),
            layer3(ln2_g), layer3(ln2_b),
            full2(lnf_g2), full2(lnf_b2),
        ],
        out_specs=pl.BlockSpec((N * L, D), lambda i: (0, 0)),
        out_shape=jax.ShapeDtypeStruct((N * L, D), jnp.float32),
    )(x2, Wq, bq, Wk, bk, Wv, bv, Wo, bo, ln1_g, ln1_b,
      W1, b1, W2, b2, ln2_g, ln2_b, lnf_g2, lnf_b2)
    return out.reshape(N, L, D)


# speedup vs baseline: 1.1927x; 1.1927x over previous
import jax
import jax.numpy as jnp
from jax.experimental import pallas as pl
from jax.experimental.pallas import tpu as pltpu

NUM_LAYERS = 2
NHEAD = 12
D_MODEL = 768
D_FFN = 1024
HEAD_DIM = D_MODEL // NHEAD
SEQ = 2048
TILE = 1024
NTILES = SEQ // TILE
GW = 256
GRP = D_MODEL // GW


def _ln(x, g, b, eps=1e-5):
    mu = jnp.mean(x, axis=-1, keepdims=True)
    xc = x - mu
    var = jnp.mean(xc * xc, axis=-1, keepdims=True)
    return xc * jax.lax.rsqrt(var + eps) * g + b


def _bf(t):
    return t.astype(jnp.bfloat16)


def _dot(a, b):
    return jax.lax.dot_general(_bf(a), _bf(b), (((1,), (0,)), ((), ())),
                               preferred_element_type=jnp.float32)


def _dot_tn(a, b):
    return jax.lax.dot_general(_bf(a), _bf(b), (((0,), (0,)), ((), ())),
                               preferred_element_type=jnp.float32)


def _feat(t):
    return jnp.where(t > 0, t + 1.0, jnp.exp(t))


def _block_kernel(x_ref, Wq_ref, bq_ref, Wk_ref, bk_ref, Wv_ref, bv_ref,
                  Wo_ref, bo_ref, ln1g_ref, ln1b_ref, W1_ref, b1_ref,
                  W2_ref, b2_ref, ln2g_ref, ln2b_ref, lnfg_ref, lnfb_ref,
                  out_ref):
    i = pl.program_id(0)

    @pl.when(i == 0)
    def _():
        out_ref[...] = x_ref[...]

    def pass_a(t, carry):
        KV, Ksum = carry
        xt = _bf(out_ref[pl.ds(t * TILE, TILE), :])
        Kt = _feat(_dot(xt, Wk_ref[0]) + bk_ref[0])
        vt = _bf(_dot(xt, Wv_ref[0]) + bv_ref[0])
        Kt16 = _bf(Kt)
        KV = [KV[g] + _dot_tn(Kt16[:, g * GW:(g + 1) * GW],
                              vt[:, g * GW:(g + 1) * GW])
              for g in range(GRP)]
        return KV, Ksum + jnp.sum(Kt, axis=0, keepdims=True)

    KV0 = [jnp.zeros((GW, GW), jnp.float32) for _ in range(GRP)]
    Ks0 = jnp.zeros((1, D_MODEL), jnp.float32)
    carry = (KV0, Ks0)
    for t in range(NTILES):
        carry = pass_a(t, carry)
    KV, Ksum = carry

    r = jax.lax.broadcasted_iota(jnp.int32, (GW, GW), 0)
    c = jax.lax.broadcasted_iota(jnp.int32, (GW, GW), 1)
    gmask = r // HEAD_DIM == c // HEAD_DIM
    KVm = [_bf(jnp.where(gmask, KV[g], 0.0)) for g in range(GRP)]
    hd = jax.lax.broadcasted_iota(jnp.int32, (D_MODEL, NHEAD), 0)
    hh = jax.lax.broadcasted_iota(jnp.int32, (D_MODEL, NHEAD), 1)
    Bh = (hd // HEAD_DIM == hh).astype(jnp.bfloat16)

    def pass_b(t):
        xt = out_ref[pl.ds(t * TILE, TILE), :]
        Qt = _feat(_dot(xt, Wq_ref[0]) + bq_ref[0])
        Qt16 = _bf(Qt)
        num = jnp.concatenate(
            [_dot(Qt16[:, g * GW:(g + 1) * GW], KVm[g]) for g in range(GRP)],
            axis=1)
        den_h = _dot(Qt * Ksum, Bh)
        den = _dot(den_h, Bh.T)
        at = num / (den + 1e-6)
        at = _dot(at, Wo_ref[0]) + bo_ref[0]
        ht = _ln(xt + at, ln1g_ref[0], ln1b_ref[0])
        yt = jnp.maximum(_dot(ht, W1_ref[0]) + b1_ref[0], 0.0)
        yt = _dot(yt, W2_ref[0]) + b2_ref[0]
        x2t = _ln(ht + yt, ln2g_ref[0], ln2b_ref[0])

        @pl.when(i == NUM_LAYERS - 1)
        def _():
            out_ref[pl.ds(t * TILE, TILE), :] = _ln(x2t, lnfg_ref[0],
                                                    lnfb_ref[0])

        @pl.when(i != NUM_LAYERS - 1)
        def _():
            out_ref[pl.ds(t * TILE, TILE), :] = x2t

    for t in range(NTILES):
        pass_b(t)


@jax.jit
def kernel(x, Wq, bq, Wk, bk, Wv, bv, Wo, bo, ln1_g, ln1_b, W1, b1, W2, b2,
           ln2_g, ln2_b, lnf_g, lnf_b):
    N, L, D = x.shape
    x2 = x.reshape(N * L, D)
    r2 = lambda t: t.reshape(NUM_LAYERS, 1, t.shape[-1])
    bq, bk, bv, bo = r2(bq), r2(bk), r2(bv), r2(bo)
    ln1_g, ln1_b, ln2_g, ln2_b = r2(ln1_g), r2(ln1_b), r2(ln2_g), r2(ln2_b)
    b1, b2 = r2(b1), r2(b2)
    lnf_g2 = lnf_g.reshape(1, D)
    lnf_b2 = lnf_b.reshape(1, D)

    full2 = lambda t: pl.BlockSpec(t.shape, lambda i: (0, 0))
    layer3 = lambda t: pl.BlockSpec((1,) + t.shape[1:], lambda i: (i, 0, 0))

    out = pl.pallas_call(
        _block_kernel,
        grid=(NUM_LAYERS,),
        in_specs=[
            full2(x2),
            layer3(Wq), layer3(bq), layer3(Wk), layer3(bk),
            layer3(Wv), layer3(bv), layer3(Wo), layer3(bo),
            layer3(ln1_g), layer3(ln1_b),
            layer3(W1), layer3(b1), layer3(W2), layer3(b2),
            layer3(ln2_g), layer3(ln2_b),
            full2(lnf_g2), full2(lnf_b2),
        ],
        out_specs=pl.BlockSpec((N * L, D), lambda i: (0, 0)),
        out_shape=jax.ShapeDtypeStruct((N * L, D), jnp.float32),
    )(x2, Wq, bq, Wk, bk, Wv, bv, Wo, bo, ln1_g, ln1_b,
      W1, b1, W2, b2, ln2_g, ln2_b, lnf_g2, lnf_b2)
    return out.reshape(N, L, D)
